# baseline (speedup 1.0000x reference)
import jax
import jax.numpy as jnp
from jax import lax
from jax.experimental import pallas as pl
from jax.experimental.pallas import tpu as pltpu
from jax.experimental.pallas import tpu_sc as plsc

B = 16384
_NC = 2
_NS = 16
_NW = _NC * _NS
_ROWS_PER_W = B // _NW
_CHUNK = 128
_NCHUNK = _ROWS_PER_W // _CHUNK
_EW = 16
_NROUTE, _NNODE, _NWT = 500, 3200, 24


_CTAB = 3744


def _sc_gather_body(rid_hbm, nid_hbm, wid_hbm, ctab_hbm,
                    x_out,
                    ctab_sp,
                    ridx_v, nidx_v, widx_v, rrows_v, nrows_v, wrows_v,
                    sem_idx, sem_g, sem_st, sem_tab):
    sid = lax.axis_index("s")
    wid = sid * _NC + lax.axis_index("c")
    base = wid * _ROWS_PER_W
    sl = pl.ds(base, _ROWS_PER_W)
    idx_loads = [pltpu.async_copy(h.at[sl], v, sem_idx)
                 for h, v in ((rid_hbm, ridx_v), (nid_hbm, nidx_v),
                              (wid_hbm, widx_v))]
    trows = _CTAB // _NS
    tsl = pl.ds(sid * trows, trows)
    pltpu.async_copy(ctab_hbm.at[tsl], ctab_sp.at[tsl], sem_tab).wait()
    for c in idx_loads:
        c.wait()
    def _rebase(i, carry):
        isl = pl.ds(i * 16, 16)
        nidx_v[isl] = nidx_v[isl] + _NROUTE
        widx_v[isl] = widx_v[isl] + (_NROUTE + _NNODE)
        return carry
    lax.fori_loop(0, _ROWS_PER_W // 16, _rebase, 0, unroll=False)
    plsc.subcore_barrier()
    idxs = (ridx_v, nidx_v, widx_v)
    rows = (rrows_v, nrows_v, wrows_v)

    def _chunk(c, carry):
        csl = pl.ds(c * _CHUNK, _CHUNK)
        osl = pl.ds(base + c * _CHUNK, _CHUNK)
        gathers = [pltpu.async_copy(ctab_sp.at[idxs[k].at[csl]],
                                    rows[k].at[csl], sem_g)
                   for k in range(3)]
        for g in gathers:
            g.wait()
        stores = [pltpu.async_copy(rows[k].at[csl],
                                   x_out.at[osl, pl.ds(k * _EW, _EW)],
                                   sem_st)
                  for k in range(3)]
        for s in stores:
            s.wait()
        return carry
    lax.fori_loop(0, _NCHUNK, _chunk, 0, unroll=False)


def _sc_gather(route_id, node_id, wt_id, ctab):
    mesh = plsc.VectorSubcoreMesh(core_axis_name="c", subcore_axis_name="s")
    idx_t = pltpu.VMEM((_ROWS_PER_W,), jnp.int32)
    rows_t = pltpu.VMEM((_ROWS_PER_W, _EW), jnp.float32)
    f = pl.kernel(
        _sc_gather_body,
        out_type=jax.ShapeDtypeStruct((B, 128), jnp.float32),
        mesh=mesh,
        scratch_types=[
            pltpu.VMEM_SHARED((_CTAB, _EW), jnp.float32),
            idx_t, idx_t, idx_t, rows_t, rows_t, rows_t,
            pltpu.SemaphoreType.DMA,
            pltpu.SemaphoreType.DMA,
            pltpu.SemaphoreType.DMA,
            pltpu.SemaphoreType.DMA,
        ],
        compiler_params=pltpu.CompilerParams(use_tc_tiling_on_sc=False),
    )
    return f(route_id, node_id, wt_id, ctab)


_R = 8192


def _mlp_body(dense_t, x, w1d, w1e, b1, w2t, b2, w3, b3, out):
    f32 = jnp.float32
    bf16 = jnp.bfloat16
    lanes = lax.broadcasted_iota(jnp.int32, (1, 128), 1)
    xc = jnp.where(lanes < 3 * _EW, x[...], 0.0)
    h = (lax.dot_general(dense_t[...], w1d[...], (((0,), (0,)), ((), ())),
                         preferred_element_type=f32)
         + jnp.dot(xc.astype(bf16), w1e[...].astype(bf16),
                   preferred_element_type=f32) + b1[...])
    h = jnp.maximum(h, 0.0)
    h = jnp.maximum(jnp.dot(h.astype(bf16), w2t[...].astype(bf16),
                            preferred_element_type=f32) + b2[...], 0.0)
    out[...] = lax.dot_general(w3[...], h, (((1,), (1,)), ((), ())),
                               preferred_element_type=f32) + b3[...]


def _tc_mlp(dense_t, x, w1d, w1e, b1, w2t, b2, w3, b3):
    grid = (B // _R,)
    row = lambda i: (i, 0)
    rep = lambda i: (0, 0)
    col = lambda i: (0, i)
    return pl.pallas_call(
        _mlp_body,
        grid=grid,
        in_specs=[
            pl.BlockSpec((6, _R), col),
            pl.BlockSpec((_R, 128), row),
            pl.BlockSpec((6, 128), rep),
            pl.BlockSpec((128, 128), rep),
            pl.BlockSpec((1, 128), rep),
            pl.BlockSpec((128, 64), rep),
            pl.BlockSpec((1, 64), rep),
            pl.BlockSpec((1, 64), rep),
            pl.BlockSpec((1, 1), rep),
        ],
        out_specs=pl.BlockSpec((1, _R), col),
        out_shape=jax.ShapeDtypeStruct((1, B), jnp.float32),
    )(dense_t, x, w1d, w1e, b1, w2t, b2, w3, b3)


def kernel(route_id, node_id, weekday_timegroup, dense_feats, route_table,
           node_table, wt_table, W1, b1, W2, b2, W3, b3):
    rtab = jnp.pad(route_table, ((0, 0), (0, _EW - 8)))
    wtab = jnp.pad(wt_table, ((0, 0), (0, _EW - 4), ))
    ctab = jnp.concatenate(
        [rtab, node_table, wtab,
         jnp.zeros((_CTAB - _NROUTE - _NNODE - _NWT, _EW), jnp.float32)],
        axis=0)

    x = _sc_gather(route_id.astype(jnp.int32), node_id.astype(jnp.int32),
                   weekday_timegroup.astype(jnp.int32), ctab)

    w1e = jnp.zeros((128, 128), jnp.float32)
    w1e = w1e.at[0:8, :].set(W1[:, 6:14].T)
    w1e = w1e.at[16:32, :].set(W1[:, 14:30].T)
    w1e = w1e.at[32:36, :].set(W1[:, 30:34].T)
    w1d = W1[:, 0:6].T

    out = _tc_mlp(dense_feats.T, x, w1d, w1e, b1.reshape(1, 128), W2.T,
                  b2.reshape(1, 64), W3, b3.reshape(1, 1))
    return out.reshape(B)


# speedup vs baseline: 1.0430x; 1.0430x over previous
import jax
import jax.numpy as jnp
from jax import lax
from jax.experimental import pallas as pl
from jax.experimental.pallas import tpu as pltpu
from jax.experimental.pallas import tpu_sc as plsc

B = 16384
_NC = 2
_NS = 16
_NW = _NC * _NS
_ROWS_PER_W = B // _NW
_CHUNK = 128
_NCHUNK = _ROWS_PER_W // _CHUNK
_EW = 16
_NROUTE, _NNODE, _NWT = 500, 3200, 24


_CTAB = 3744


def _sc_gather_body(rid_hbm, nid_hbm, wid_hbm, ctab_hbm,
                    x_out,
                    ctab_sp,
                    ridx_v, nidx_v, widx_v, rrows_v, nrows_v, wrows_v,
                    sem_idx, sem_g, sem_st, sem_tab):
    sid = lax.axis_index("s")
    wid = sid * _NC + lax.axis_index("c")
    base = wid * _ROWS_PER_W
    sl = pl.ds(base, _ROWS_PER_W)
    idx_loads = [pltpu.async_copy(h.at[sl], v, sem_idx)
                 for h, v in ((rid_hbm, ridx_v), (nid_hbm, nidx_v),
                              (wid_hbm, widx_v))]
    trows = _CTAB // _NS
    tsl = pl.ds(sid * trows, trows)
    pltpu.async_copy(ctab_hbm.at[tsl], ctab_sp.at[tsl], sem_tab).wait()
    for c in idx_loads:
        c.wait()
    for off, v in ((_NROUTE, nidx_v), (_NROUTE + _NNODE, widx_v)):
        for i in range(_ROWS_PER_W // 16):
            isl = pl.ds(i * 16, 16)
            v[isl] = v[isl] + off
    plsc.subcore_barrier()
    idxs = (ridx_v, nidx_v, widx_v)
    rows = (rrows_v, nrows_v, wrows_v)
    gathers = []
    for c in range(_NCHUNK):
        csl = pl.ds(c * _CHUNK, _CHUNK)
        for k in range(3):
            gathers.append(pltpu.async_copy(ctab_sp.at[idxs[k].at[csl]],
                                            rows[k].at[csl], sem_g))
    stores = []
    for c in range(_NCHUNK):
        csl = pl.ds(c * _CHUNK, _CHUNK)
        osl = pl.ds(base + c * _CHUNK, _CHUNK)
        for k in range(3):
            gathers[c * 3 + k].wait()
            stores.append(pltpu.async_copy(
                rows[k].at[csl], x_out.at[osl, pl.ds(k * _EW, _EW)],
                sem_st))
    for c in stores:
        c.wait()


def _sc_gather(route_id, node_id, wt_id, ctab):
    mesh = plsc.VectorSubcoreMesh(core_axis_name="c", subcore_axis_name="s")
    idx_t = pltpu.VMEM((_ROWS_PER_W,), jnp.int32)
    rows_t = pltpu.VMEM((_ROWS_PER_W, _EW), jnp.float32)
    f = pl.kernel(
        _sc_gather_body,
        out_type=jax.ShapeDtypeStruct((B, 128), jnp.float32),
        mesh=mesh,
        scratch_types=[
            pltpu.VMEM_SHARED((_CTAB, _EW), jnp.float32),
            idx_t, idx_t, idx_t, rows_t, rows_t, rows_t,
            pltpu.SemaphoreType.DMA,
            pltpu.SemaphoreType.DMA,
            pltpu.SemaphoreType.DMA,
            pltpu.SemaphoreType.DMA,
        ],
        compiler_params=pltpu.CompilerParams(use_tc_tiling_on_sc=False),
    )
    return f(route_id, node_id, wt_id, ctab)


_R = 8192


def _mlp_body(dense_t, x, w1d, w1e, b1, w2t, b2, w3, b3, out):
    f32 = jnp.float32
    bf16 = jnp.bfloat16
    lanes = lax.broadcasted_iota(jnp.int32, (1, 128), 1)
    xc = jnp.where(lanes < 3 * _EW, x[...], 0.0)
    h = (lax.dot_general(dense_t[...], w1d[...], (((0,), (0,)), ((), ())),
                         preferred_element_type=f32)
         + jnp.dot(xc.astype(bf16), w1e[...].astype(bf16),
                   preferred_element_type=f32) + b1[...])
    h = jnp.maximum(h, 0.0)
    h = jnp.maximum(jnp.dot(h.astype(bf16), w2t[...].astype(bf16),
                            preferred_element_type=f32) + b2[...], 0.0)
    out[...] = lax.dot_general(w3[...], h, (((1,), (1,)), ((), ())),
                               preferred_element_type=f32) + b3[...]


def _tc_mlp(dense_t, x, w1d, w1e, b1, w2t, b2, w3, b3):
    grid = (B // _R,)
    row = lambda i: (i, 0)
    rep = lambda i: (0, 0)
    col = lambda i: (0, i)
    return pl.pallas_call(
        _mlp_body,
        grid=grid,
        in_specs=[
            pl.BlockSpec((6, _R), col),
            pl.BlockSpec((_R, 128), row),
            pl.BlockSpec((6, 128), rep),
            pl.BlockSpec((128, 128), rep),
            pl.BlockSpec((1, 128), rep),
            pl.BlockSpec((128, 64), rep),
            pl.BlockSpec((1, 64), rep),
            pl.BlockSpec((1, 64), rep),
            pl.BlockSpec((1, 1), rep),
        ],
        out_specs=pl.BlockSpec((1, _R), col),
        out_shape=jax.ShapeDtypeStruct((1, B), jnp.float32),
    )(dense_t, x, w1d, w1e, b1, w2t, b2, w3, b3)


def kernel(route_id, node_id, weekday_timegroup, dense_feats, route_table,
           node_table, wt_table, W1, b1, W2, b2, W3, b3):
    rtab = jnp.pad(route_table, ((0, 0), (0, _EW - 8)))
    wtab = jnp.pad(wt_table, ((0, 0), (0, _EW - 4), ))
    ctab = jnp.concatenate(
        [rtab, node_table, wtab,
         jnp.zeros((_CTAB - _NROUTE - _NNODE - _NWT, _EW), jnp.float32)],
        axis=0)

    x = _sc_gather(route_id.astype(jnp.int32), node_id.astype(jnp.int32),
                   weekday_timegroup.astype(jnp.int32), ctab)

    w1e = jnp.zeros((128, 128), jnp.float32)
    w1e = w1e.at[0:8, :].set(W1[:, 6:14].T)
    w1e = w1e.at[16:32, :].set(W1[:, 14:30].T)
    w1e = w1e.at[32:36, :].set(W1[:, 30:34].T)
    w1d = W1[:, 0:6].T

    out = _tc_mlp(dense_feats.T, x, w1d, w1e, b1.reshape(1, 128), W2.T,
                  b2.reshape(1, 64), W3, b3.reshape(1, 1))
    return out.reshape(B)
